# baseline (speedup 1.0000x reference)
import functools

import jax
import jax.numpy as jnp
from jax import lax
from jax.experimental import pallas as pl
from jax.experimental.pallas import tpu as pltpu
from jax.experimental.pallas import tpu_sc as plsc

N = 8192
D = 64
B = 256
K = 8
R = 256
M = N * K

NC = 2
NS = 16
L = 16
NW = NC * NS
XS_PER = B // NW
CO_PER = D // NW
CHUNK = 2048


def _topk_body(cb_ref, call_ref, out_ref, dref, sv_ref, si_ref):
    cb = cb_ref[...]
    call = call_ref[...]
    dot = lax.dot_general(cb, call, (((0,), (0,)), ((), ())),
                          preferred_element_type=jnp.float32)
    xx = jnp.sum(jnp.square(call), axis=0)[None, :]
    yy = jnp.sum(jnp.square(cb), axis=0)[:, None]
    dref[...] = jnp.maximum(dot * (-2.0) + xx + yy, 0.0)

    INF = jnp.float32(jnp.inf)
    NS_ = N // 128
    RS = 16

    def subblock(r16, carry):
        rows = pl.ds(r16 * RS, RS)
        v1 = jnp.full((RS, 128), INF, dtype=jnp.float32)
        v2 = v1
        v3 = v1
        v4 = v1
        i1 = jnp.full((RS, 128), NS_, dtype=jnp.int32)
        i2 = i1
        i3 = i1
        for m in range(NS_):
            e = dref[rows, m * 128:(m + 1) * 128]
            mc = jnp.full((RS, 128), m, dtype=jnp.int32)
            b1 = e < v1
            b2 = e < v2
            b3 = e < v3
            b4 = e < v4
            nv1 = jnp.where(b1, e, v1)
            ni1 = jnp.where(b1, mc, i1)
            nv2 = jnp.where(b1, v1, jnp.where(b2, e, v2))
            ni2 = jnp.where(b1, i1, jnp.where(b2, mc, i2))
            nv3 = jnp.where(b2, v2, jnp.where(b3, e, v3))
            ni3 = jnp.where(b2, i2, jnp.where(b3, mc, i3))
            v4 = jnp.where(b3, v3, jnp.where(b4, e, v4))
            v1, i1, v2, i2, v3, i3 = nv1, ni1, nv2, ni2, nv3, ni3
        sv_ref[0, rows, :] = v1
        sv_ref[1, rows, :] = v2
        sv_ref[2, rows, :] = v3
        sv_ref[3, rows, :] = v4
        si_ref[0, rows, :] = i1
        si_ref[1, rows, :] = i2
        si_ref[2, rows, :] = i3
        return carry

    lax.fori_loop(0, R // RS, subblock, 0)

    QS = R // 4
    BIG = jnp.int32(N * 2)
    lane_q = lax.broadcasted_iota(jnp.int32, (QS, 128), 1)
    oks = []
    for q in range(4):
        rows = slice(q * QS, (q + 1) * QS)
        v1 = sv_ref[0, rows, :]
        v2 = sv_ref[1, rows, :]
        v3 = sv_ref[2, rows, :]
        v4 = sv_ref[3, rows, :]
        g1 = si_ref[0, rows, :] * 128 + lane_q
        g2 = si_ref[1, rows, :] * 128 + lane_q
        g3 = si_ref[2, rows, :] * 128 + lane_q
        cols = []
        m_k = None
        for _ in range(K):
            vm = jnp.minimum(jnp.minimum(v1, v2), v3)
            m_k = jnp.min(vm, axis=1, keepdims=True)
            cand = jnp.minimum(
                jnp.minimum(jnp.where(v1 == m_k, g1, BIG),
                            jnp.where(v2 == m_k, g2, BIG)),
                jnp.where(v3 == m_k, g3, BIG))
            idx_k = jnp.min(cand, axis=1, keepdims=True)
            v1 = jnp.where(g1 == idx_k, INF, v1)
            v2 = jnp.where(g2 == idx_k, INF, v2)
            v3 = jnp.where(g3 == idx_k, INF, v3)
            cols.append(idx_k)
        out_ref[rows, :] = jnp.concatenate(cols, axis=1)
        oks.append(jnp.logical_not(jnp.any(v4 <= m_k)))

    ok = jnp.logical_and(jnp.logical_and(oks[0], oks[1]),
                         jnp.logical_and(oks[2], oks[3]))

    @pl.when(jnp.logical_not(ok))
    def _fallback():
        dd = dref[...]
        io = lax.broadcasted_iota(jnp.int32, (R, N), 1)
        fcols = []
        for _ in range(K):
            mm = jnp.min(dd, axis=1, keepdims=True)
            ii = jnp.min(jnp.where(dd == mm, io, N), axis=1,
                         keepdims=True)
            fcols.append(ii)
            dd = jnp.where(io == ii, INF, dd)
        out_ref[...] = jnp.concatenate(fcols, axis=1)


def _neighbor_indices(coordinates):
    return pl.pallas_call(
        _topk_body,
        grid=(N // R,),
        in_specs=[
            pl.BlockSpec((D, R), lambda i: (0, i)),
            pl.BlockSpec((D, N), lambda i: (0, 0)),
        ],
        out_specs=pl.BlockSpec((R, K), lambda i: (i, 0)),
        out_shape=jax.ShapeDtypeStruct((N, K), jnp.int32),
        scratch_shapes=[pltpu.VMEM((R, N), jnp.float32),
                        pltpu.VMEM((4, R, 128), jnp.float32),
                        pltpu.VMEM((3, R, 128), jnp.int32)],
    )(coordinates, coordinates)


NCH = M // CHUNK


def _gather_body(xs_hbm, co_hbm, flat_hbm, oxs_hbm, oco_hbm,
                 xsrows, corows, idxbuf, obx, obc, isem, osem):
    c = lax.axis_index("c")
    s = lax.axis_index("s")
    wid = s * NC + c
    pltpu.sync_copy(xs_hbm.at[pl.ds(wid * XS_PER, XS_PER)], xsrows)
    pltpu.sync_copy(co_hbm.at[pl.ds(wid * CO_PER, CO_PER)], corows)

    def idx_start(ci, b):
        pltpu.async_copy(flat_hbm.at[pl.ds(ci * CHUNK, CHUNK)],
                         idxbuf.at[b], isem.at[b])

    def idx_wait(b):
        pltpu.make_async_copy(flat_hbm.at[pl.ds(0, CHUNK)],
                              idxbuf.at[b], isem.at[b]).wait()

    def gather_chunk(b):
        @plsc.parallel_loop(0, CHUNK // L, unroll=4)
        def step(j):
            iv = idxbuf[b, pl.ds(j * L, L)]
            for r in range(XS_PER):
                rv = jnp.full((L,), r, dtype=jnp.int32)
                obx[b, r, pl.ds(j * L, L)] = plsc.load_gather(xsrows, [rv, iv])
            for r in range(CO_PER):
                rv = jnp.full((L,), r, dtype=jnp.int32)
                obc[b, r, pl.ds(j * L, L)] = plsc.load_gather(corows, [rv, iv])

    def out_start(ci, b):
        base = ci * CHUNK
        for r in range(XS_PER):
            pltpu.async_copy(obx.at[b, r],
                             oxs_hbm.at[wid * XS_PER + r, pl.ds(base, CHUNK)],
                             osem.at[b])
        for r in range(CO_PER):
            pltpu.async_copy(obc.at[b, r],
                             oco_hbm.at[wid * CO_PER + r, pl.ds(base, CHUNK)],
                             osem.at[b])

    def out_wait(b):
        for r in range(XS_PER):
            pltpu.make_async_copy(obx.at[b, r],
                                  oxs_hbm.at[wid * XS_PER + r, pl.ds(0, CHUNK)],
                                  osem.at[b]).wait()
        for r in range(CO_PER):
            pltpu.make_async_copy(obc.at[b, r],
                                  oco_hbm.at[wid * CO_PER + r, pl.ds(0, CHUNK)],
                                  osem.at[b]).wait()

    idx_start(0, 0)
    idx_start(1, 1)
    idx_wait(0)
    v0 = idxbuf[0, pl.ds(0, L)]
    idxbuf[0, pl.ds(0, L)] = jnp.where(
        lax.iota(jnp.int32, L) == 0, jnp.int32(0), v0)
    gather_chunk(0)
    idx_start(2, 0)
    out_start(0, 0)
    idx_wait(1)
    gather_chunk(1)
    idx_start(3, 1)
    out_start(1, 1)

    def pair(g, carry):
        ci0 = g * 2
        out_wait(0)
        idx_wait(0)
        gather_chunk(0)
        idx_start(ci0 + 2, 0)
        out_start(ci0, 0)
        out_wait(1)
        idx_wait(1)
        gather_chunk(1)
        idx_start(ci0 + 3, 1)
        out_start(ci0 + 1, 1)
        return carry

    lax.fori_loop(1, NCH // 2 - 1, pair, 0)

    out_wait(0)
    idx_wait(0)
    gather_chunk(0)
    out_start(NCH - 2, 0)
    out_wait(1)
    idx_wait(1)
    gather_chunk(1)
    out_start(NCH - 1, 1)
    out_wait(0)
    out_wait(1)


@functools.lru_cache(maxsize=1)
def _gather_kernel():
    return pl.kernel(
        _gather_body,
        out_type=(
            jax.ShapeDtypeStruct((B, M), jnp.float32),
            jax.ShapeDtypeStruct((D, M), jnp.float32),
        ),
        mesh=plsc.VectorSubcoreMesh(
            core_axis_name="c", subcore_axis_name="s",
            num_cores=NC, num_subcores=NS,
        ),
        compiler_params=pltpu.CompilerParams(
            use_tc_tiling_on_sc=False, needs_layout_passes=False),
        scratch_types=[
            pltpu.VMEM((XS_PER, N), jnp.float32),
            pltpu.VMEM((CO_PER, N), jnp.float32),
            pltpu.VMEM((2, CHUNK), jnp.int32),
            pltpu.VMEM((2, XS_PER, CHUNK), jnp.float32),
            pltpu.VMEM((2, CO_PER, CHUNK), jnp.float32),
            pltpu.SemaphoreType.DMA((2,)),
            pltpu.SemaphoreType.DMA((2,)),
        ],
    )


def kernel(xs, coordinates):
    idx = _neighbor_indices(coordinates)
    flat = idx.reshape(-1)
    oxs, oco = _gather_kernel()(xs, coordinates, flat)
    return oxs[:, None, :, None], oco[:, None, :, None]


# speedup vs baseline: 1.0619x; 1.0619x over previous
import functools

import jax
import jax.numpy as jnp
from jax import lax
from jax.experimental import pallas as pl
from jax.experimental.pallas import tpu as pltpu
from jax.experimental.pallas import tpu_sc as plsc

N = 8192
D = 64
B = 256
K = 8
R = 512
M = N * K

NC = 2
NS = 16
L = 16
NW = NC * NS
XS_PER = B // NW
CO_PER = D // NW
CHUNK = 2048


def _topk_body(cb_ref, call_ref, out_ref, dref, sv_ref, si_ref):
    cb = cb_ref[...]
    call = call_ref[...]
    dot = lax.dot_general(cb, call, (((0,), (0,)), ((), ())),
                          preferred_element_type=jnp.float32)
    xx = jnp.sum(jnp.square(call), axis=0)[None, :]
    yy = jnp.sum(jnp.square(cb), axis=0)[:, None]
    dref[...] = jnp.maximum(dot * (-2.0) + xx + yy, 0.0)

    INF = jnp.float32(jnp.inf)
    NS_ = N // 128
    RS = 16

    def subblock(r16, carry):
        rows = pl.ds(r16 * RS, RS)
        v1 = jnp.full((RS, 128), INF, dtype=jnp.float32)
        v2 = v1
        v3 = v1
        v4 = v1
        i1 = jnp.full((RS, 128), NS_, dtype=jnp.int32)
        i2 = i1
        i3 = i1
        for m in range(NS_):
            e = dref[rows, m * 128:(m + 1) * 128]
            mc = jnp.full((RS, 128), m, dtype=jnp.int32)
            b1 = e < v1
            b2 = e < v2
            b3 = e < v3
            b4 = e < v4
            nv1 = jnp.where(b1, e, v1)
            ni1 = jnp.where(b1, mc, i1)
            nv2 = jnp.where(b1, v1, jnp.where(b2, e, v2))
            ni2 = jnp.where(b1, i1, jnp.where(b2, mc, i2))
            nv3 = jnp.where(b2, v2, jnp.where(b3, e, v3))
            ni3 = jnp.where(b2, i2, jnp.where(b3, mc, i3))
            v4 = jnp.where(b3, v3, jnp.where(b4, e, v4))
            v1, i1, v2, i2, v3, i3 = nv1, ni1, nv2, ni2, nv3, ni3
        sv_ref[0, rows, :] = v1
        sv_ref[1, rows, :] = v2
        sv_ref[2, rows, :] = v3
        sv_ref[3, rows, :] = v4
        si_ref[0, rows, :] = i1
        si_ref[1, rows, :] = i2
        si_ref[2, rows, :] = i3
        return carry

    lax.fori_loop(0, R // RS, subblock, 0)

    QS = R // 4
    BIG = jnp.int32(N * 2)
    lane_q = lax.broadcasted_iota(jnp.int32, (QS, 128), 1)
    oks = []
    for q in range(4):
        rows = slice(q * QS, (q + 1) * QS)
        v1 = sv_ref[0, rows, :]
        v2 = sv_ref[1, rows, :]
        v3 = sv_ref[2, rows, :]
        v4 = sv_ref[3, rows, :]
        g1 = si_ref[0, rows, :] * 128 + lane_q
        g2 = si_ref[1, rows, :] * 128 + lane_q
        g3 = si_ref[2, rows, :] * 128 + lane_q
        cols = []
        m_k = None
        for _ in range(K):
            vm = jnp.minimum(jnp.minimum(v1, v2), v3)
            m_k = jnp.min(vm, axis=1, keepdims=True)
            cand = jnp.minimum(
                jnp.minimum(jnp.where(v1 == m_k, g1, BIG),
                            jnp.where(v2 == m_k, g2, BIG)),
                jnp.where(v3 == m_k, g3, BIG))
            idx_k = jnp.min(cand, axis=1, keepdims=True)
            v1 = jnp.where(g1 == idx_k, INF, v1)
            v2 = jnp.where(g2 == idx_k, INF, v2)
            v3 = jnp.where(g3 == idx_k, INF, v3)
            cols.append(idx_k)
        out_ref[rows, :] = jnp.concatenate(cols, axis=1)
        oks.append(jnp.logical_not(jnp.any(v4 <= m_k)))

    ok = jnp.logical_and(jnp.logical_and(oks[0], oks[1]),
                         jnp.logical_and(oks[2], oks[3]))

    @pl.when(jnp.logical_not(ok))
    def _fallback():
        dd = dref[...]
        io = lax.broadcasted_iota(jnp.int32, (R, N), 1)
        fcols = []
        for _ in range(K):
            mm = jnp.min(dd, axis=1, keepdims=True)
            ii = jnp.min(jnp.where(dd == mm, io, N), axis=1,
                         keepdims=True)
            fcols.append(ii)
            dd = jnp.where(io == ii, INF, dd)
        out_ref[...] = jnp.concatenate(fcols, axis=1)


def _neighbor_indices(coordinates):
    return pl.pallas_call(
        _topk_body,
        grid=(N // R,),
        in_specs=[
            pl.BlockSpec((D, R), lambda i: (0, i)),
            pl.BlockSpec((D, N), lambda i: (0, 0)),
        ],
        out_specs=pl.BlockSpec((R, K), lambda i: (i, 0)),
        out_shape=jax.ShapeDtypeStruct((N, K), jnp.int32),
        scratch_shapes=[pltpu.VMEM((R, N), jnp.float32),
                        pltpu.VMEM((4, R, 128), jnp.float32),
                        pltpu.VMEM((3, R, 128), jnp.int32)],
    )(coordinates, coordinates)


NCH = M // CHUNK


def _gather_body(xs_hbm, co_hbm, flat_hbm, oxs_hbm, oco_hbm,
                 xsrows, corows, idxbuf, obx, obc, isem, osem):
    c = lax.axis_index("c")
    s = lax.axis_index("s")
    wid = s * NC + c
    pltpu.sync_copy(xs_hbm.at[pl.ds(wid * XS_PER, XS_PER)], xsrows)
    pltpu.sync_copy(co_hbm.at[pl.ds(wid * CO_PER, CO_PER)], corows)

    def idx_start(ci, b):
        pltpu.async_copy(flat_hbm.at[pl.ds(ci * CHUNK, CHUNK)],
                         idxbuf.at[b], isem.at[b])

    def idx_wait(b):
        pltpu.make_async_copy(flat_hbm.at[pl.ds(0, CHUNK)],
                              idxbuf.at[b], isem.at[b]).wait()

    def gather_chunk(b):
        @plsc.parallel_loop(0, CHUNK // L, unroll=4)
        def step(j):
            iv = idxbuf[b, pl.ds(j * L, L)]
            for r in range(XS_PER):
                rv = jnp.full((L,), r, dtype=jnp.int32)
                obx[b, r, pl.ds(j * L, L)] = plsc.load_gather(xsrows, [rv, iv])
            for r in range(CO_PER):
                rv = jnp.full((L,), r, dtype=jnp.int32)
                obc[b, r, pl.ds(j * L, L)] = plsc.load_gather(corows, [rv, iv])

    def out_start(ci, b):
        base = ci * CHUNK
        for r in range(XS_PER):
            pltpu.async_copy(obx.at[b, r],
                             oxs_hbm.at[wid * XS_PER + r, pl.ds(base, CHUNK)],
                             osem.at[b])
        for r in range(CO_PER):
            pltpu.async_copy(obc.at[b, r],
                             oco_hbm.at[wid * CO_PER + r, pl.ds(base, CHUNK)],
                             osem.at[b])

    def out_wait(b):
        for r in range(XS_PER):
            pltpu.make_async_copy(obx.at[b, r],
                                  oxs_hbm.at[wid * XS_PER + r, pl.ds(0, CHUNK)],
                                  osem.at[b]).wait()
        for r in range(CO_PER):
            pltpu.make_async_copy(obc.at[b, r],
                                  oco_hbm.at[wid * CO_PER + r, pl.ds(0, CHUNK)],
                                  osem.at[b]).wait()

    idx_start(0, 0)
    idx_start(1, 1)
    idx_wait(0)
    v0 = idxbuf[0, pl.ds(0, L)]
    idxbuf[0, pl.ds(0, L)] = jnp.where(
        lax.iota(jnp.int32, L) == 0, jnp.int32(0), v0)
    gather_chunk(0)
    idx_start(2, 0)
    out_start(0, 0)
    idx_wait(1)
    gather_chunk(1)
    idx_start(3, 1)
    out_start(1, 1)

    def pair(g, carry):
        ci0 = g * 2
        out_wait(0)
        idx_wait(0)
        gather_chunk(0)
        idx_start(ci0 + 2, 0)
        out_start(ci0, 0)
        out_wait(1)
        idx_wait(1)
        gather_chunk(1)
        idx_start(ci0 + 3, 1)
        out_start(ci0 + 1, 1)
        return carry

    lax.fori_loop(1, NCH // 2 - 1, pair, 0)

    out_wait(0)
    idx_wait(0)
    gather_chunk(0)
    out_start(NCH - 2, 0)
    out_wait(1)
    idx_wait(1)
    gather_chunk(1)
    out_start(NCH - 1, 1)
    out_wait(0)
    out_wait(1)


@functools.lru_cache(maxsize=1)
def _gather_kernel():
    return pl.kernel(
        _gather_body,
        out_type=(
            jax.ShapeDtypeStruct((B, M), jnp.float32),
            jax.ShapeDtypeStruct((D, M), jnp.float32),
        ),
        mesh=plsc.VectorSubcoreMesh(
            core_axis_name="c", subcore_axis_name="s",
            num_cores=NC, num_subcores=NS,
        ),
        compiler_params=pltpu.CompilerParams(
            use_tc_tiling_on_sc=False, needs_layout_passes=False),
        scratch_types=[
            pltpu.VMEM((XS_PER, N), jnp.float32),
            pltpu.VMEM((CO_PER, N), jnp.float32),
            pltpu.VMEM((2, CHUNK), jnp.int32),
            pltpu.VMEM((2, XS_PER, CHUNK), jnp.float32),
            pltpu.VMEM((2, CO_PER, CHUNK), jnp.float32),
            pltpu.SemaphoreType.DMA((2,)),
            pltpu.SemaphoreType.DMA((2,)),
        ],
    )


def kernel(xs, coordinates):
    idx = _neighbor_indices(coordinates)
    flat = idx.reshape(-1)
    oxs, oco = _gather_kernel()(xs, coordinates, flat)
    return oxs[:, None, :, None], oco[:, None, :, None]
